# baseline (speedup 1.0000x reference)
import functools
import math

import jax
import jax.numpy as jnp
from jax import lax
from jax.experimental import pallas as pl
from jax.experimental.pallas import tpu as pltpu
from jax.experimental.pallas import tpu_sc as plsc

D_MODEL = 64
PADW = 128
SCALE = math.sqrt(D_MODEL)

NUM_CORES = 2
NUM_SUBCORES = 16
NW = NUM_CORES * NUM_SUBCORES

NB = 2
SEQ = 200
SPLITS = ((0, 96), (96, 104))


def _build_gather(BATCH: int, V: int):
    assert BATCH % NW == 0
    bpw = BATCH // NW
    nchunk = bpw // NB

    mesh = plsc.VectorSubcoreMesh(core_axis_name="c", subcore_axis_name="s")

    @functools.partial(
        pl.kernel,
        mesh=mesh,
        out_type=jax.ShapeDtypeStruct((BATCH, SEQ, D_MODEL), jnp.float32),
        compiler_params=pltpu.CompilerParams(use_tc_tiling_on_sc=False),
        scratch_types=[
            pltpu.VMEM((bpw, SEQ), jnp.int32),
            pltpu.VMEM((NB, SEQ, PADW), jnp.float32),
            pltpu.VMEM((NB, SEQ, PADW), jnp.float32),
            pltpu.SemaphoreType.DMA,
            pltpu.SemaphoreType.DMA,
            pltpu.SemaphoreType.DMA,
            pltpu.SemaphoreType.DMA,
        ],
    )
    def gather_kernel(idx_hbm, table_hbm, out_hbm, idx_v, rows0, rows1,
                      gsem0, gsem1, osem0, osem1):
        cid = lax.axis_index("c")
        sid = lax.axis_index("s")
        wid = sid * NUM_CORES + cid
        rows = (rows0, rows1)
        gsem = (gsem0, gsem1)
        osem = (osem0, osem1)

        pltpu.sync_copy(idx_hbm.at[pl.ds(wid * bpw, bpw)], idx_v)

        def fire(ci, p):
            for s in range(NB):
                for off, ln in SPLITS:
                    pltpu.async_copy(
                        table_hbm.at[idx_v.at[ci * NB + s, pl.ds(off, ln)]],
                        rows[p].at[s, pl.ds(off, ln)],
                        gsem[p],
                    )

        def drain_gathers(p):
            for s in range(NB):
                for off, ln in SPLITS:
                    pltpu.make_async_copy(
                        table_hbm.at[idx_v.at[0, pl.ds(0, ln)]],
                        rows[p].at[s, pl.ds(off, ln)],
                        gsem[p],
                    ).wait()

        def put_out(ci, p):
            pltpu.async_copy(
                rows[p].at[:, :, pl.ds(0, D_MODEL)],
                out_hbm.at[pl.ds(wid * bpw + ci * NB, NB)],
                osem[p],
            )

        def drain_out(p):
            pltpu.make_async_copy(
                rows[p].at[:, :, pl.ds(0, D_MODEL)],
                out_hbm.at[pl.ds(0, NB)],
                osem[p],
            ).wait()

        fire(0, 0)

        def pair_body(i, carry):
            for p in (0, 1):
                ci = 2 * i + p
                nci = ci + 1

                @pl.when(nci < nchunk)
                def _():
                    @pl.when(ci >= 1)
                    def _():
                        drain_out(1 - p)
                    fire(nci, 1 - p)

                drain_gathers(p)
                put_out(ci, p)
            return carry

        lax.fori_loop(0, nchunk // 2, pair_body, 0, unroll=False)
        drain_out(0)
        drain_out(1)

    return gather_kernel


def kernel(x, table):
    BATCH = x.shape[0]
    V = table.shape[0]
    table_pad = jnp.pad(table * jnp.float32(SCALE),
                        ((0, 0), (0, PADW - D_MODEL)))
    return _build_gather(BATCH, V)(x, table_pad)


# speedup vs baseline: 1.3931x; 1.3931x over previous
import functools
import math

import jax
import jax.numpy as jnp
from jax import lax
from jax.experimental import pallas as pl
from jax.experimental.pallas import tpu as pltpu
from jax.experimental.pallas import tpu_sc as plsc

D_MODEL = 64
PADW = 128
SCALE = math.sqrt(D_MODEL)

NUM_CORES = 2
NUM_SUBCORES = 16
NW = NUM_CORES * NUM_SUBCORES

SEQ = 200
SEQP = 256
GRPB = 64
RUNROLL = 8
OFFS = tuple(range(0, SEQ - 16, 16)) + (SEQ - 16,)


def _build_gather(BATCH: int, V: int):
    assert BATCH % (NW * GRPB) == 0
    bpw = BATCH // NW

    mesh = plsc.VectorSubcoreMesh(core_axis_name="c", subcore_axis_name="s")

    @functools.partial(
        pl.kernel,
        mesh=mesh,
        out_type=jax.ShapeDtypeStruct((BATCH, SEQ, D_MODEL), jnp.float32),
        scratch_types=[
            pltpu.VMEM((GRPB, SEQP), jnp.int32),
            pltpu.VMEM((208, PADW), jnp.float32),
            pltpu.VMEM((208, PADW), jnp.float32),
            pltpu.VMEM((SEQ, D_MODEL), jnp.float32),
            pltpu.SemaphoreType.DMA,
            pltpu.SemaphoreType.DMA,
            pltpu.SemaphoreType.DMA,
        ],
    )
    def gather_kernel(idx_hbm, table_hbm, out_hbm, idx_v, wide0, wide1,
                      nar, gsem0, gsem1, osem):
        cid = lax.axis_index("c")
        sid = lax.axis_index("s")
        wid = sid * NUM_CORES + cid
        wide = (wide0, wide1)
        gsem = (gsem0, gsem1)

        def load_group(g):
            pltpu.sync_copy(idx_hbm.at[pl.ds(wid * bpw + g * GRPB, GRPB)],
                            idx_v)

        def fire(ci, p):
            r = ci % GRPB
            for off in OFFS:
                ivec = idx_v[r, pl.ds(off, 16)]
                pltpu.async_copy(
                    table_hbm.at[ivec],
                    wide[p].at[pl.ds(off, 16)],
                    gsem[p],
                )

        def drain_gathers(p):
            zvec = jnp.zeros((16,), jnp.int32)
            for off in OFFS:
                pltpu.make_async_copy(
                    table_hbm.at[zvec],
                    wide[p].at[pl.ds(off, 16)],
                    gsem[p],
                ).wait()

        def put_out(ci):
            pltpu.async_copy(nar, out_hbm.at[wid * bpw + ci], osem)

        def drain_out():
            pltpu.make_async_copy(nar, out_hbm.at[0], osem).wait()

        def scale(p):
            src = wide[p]

            def srows(r, carry):
                base = r * RUNROLL
                for k in range(RUNROLL):
                    for l in range(D_MODEL // 16):
                        nar[base + k, pl.ds(l * 16, 16)] = (
                            src[base + k, pl.ds(l * 16, 16)] * SCALE
                        )
                return carry

            lax.fori_loop(0, SEQ // RUNROLL, srows, 0, unroll=False)

        load_group(0)
        fire(0, 0)

        def pair_body(i, carry):
            for p in (0, 1):
                ci = 2 * i + p
                nci = ci + 1

                @pl.when(jnp.logical_and(nci < bpw, nci % GRPB != 0))
                def _():
                    fire(nci, 1 - p)

                drain_gathers(p)

                @pl.when(jnp.logical_and(nci < bpw, nci % GRPB == 0))
                def _():
                    load_group(nci // GRPB)
                    fire(nci, 1 - p)

                @pl.when(ci >= 1)
                def _():
                    drain_out()

                scale(p)
                put_out(ci)
            return carry

        lax.fori_loop(0, bpw // 2, pair_body, 0, unroll=False)
        drain_out()

    return gather_kernel


def kernel(x, table):
    BATCH = x.shape[0]
    V = table.shape[0]
    xp = jnp.pad(x, ((0, 0), (0, SEQP - SEQ)))
    table_pad = jnp.pad(table, ((0, 0), (0, PADW - D_MODEL)))
    return _build_gather(BATCH, V)(xp, table_pad)


# speedup vs baseline: 1.5419x; 1.1068x over previous
import functools
import math

import jax
import jax.numpy as jnp
from jax import lax
from jax.experimental import pallas as pl
from jax.experimental.pallas import tpu as pltpu
from jax.experimental.pallas import tpu_sc as plsc

D_MODEL = 64
PADW = 128
SCALE = math.sqrt(D_MODEL)

NUM_CORES = 2
NUM_SUBCORES = 16
NW = NUM_CORES * NUM_SUBCORES

SEQ = 200
SEQP = 256
GRPB = 64
RUNROLL = 8
OFFS = tuple(range(0, SEQ - 16, 16)) + (SEQ - 16,)


def _build_gather(BATCH: int, V: int):
    assert BATCH % (NW * GRPB) == 0
    bpw = BATCH // NW

    mesh = plsc.VectorSubcoreMesh(core_axis_name="c", subcore_axis_name="s")

    @functools.partial(
        pl.kernel,
        mesh=mesh,
        out_type=jax.ShapeDtypeStruct((BATCH * SEQ, D_MODEL), jnp.float32),
        scratch_types=[
            pltpu.VMEM((GRPB, SEQP), jnp.int32),
            pltpu.VMEM((208, PADW), jnp.float32),
            pltpu.VMEM((208, PADW), jnp.float32),
            pltpu.VMEM((SEQ, D_MODEL), jnp.float32),
            pltpu.VMEM((SEQ, D_MODEL), jnp.float32),
            pltpu.SemaphoreType.DMA,
            pltpu.SemaphoreType.DMA,
            pltpu.SemaphoreType.DMA,
            pltpu.SemaphoreType.DMA,
        ],
    )
    def gather_kernel(idx_hbm, table_hbm, out_hbm, idx_v, wide0, wide1,
                      nar0, nar1, gsem0, gsem1, osem0, osem1):
        cid = lax.axis_index("c")
        sid = lax.axis_index("s")
        wid = sid * NUM_CORES + cid
        wide = (wide0, wide1)
        nar = (nar0, nar1)
        gsem = (gsem0, gsem1)
        osem = (osem0, osem1)

        def load_group(g):
            pltpu.sync_copy(idx_hbm.at[pl.ds(wid * bpw + g * GRPB, GRPB)],
                            idx_v)

        def fire(ci, p):
            r = ci % GRPB
            for off in OFFS:
                ivec = idx_v[r, pl.ds(off, 16)]
                pltpu.async_copy(
                    table_hbm.at[ivec],
                    wide[p].at[pl.ds(off, 16)],
                    gsem[p],
                )

        def drain_gathers(p):
            zvec = jnp.zeros((16,), jnp.int32)
            for off in OFFS:
                pltpu.make_async_copy(
                    table_hbm.at[zvec],
                    wide[p].at[pl.ds(off, 16)],
                    gsem[p],
                ).wait()

        def put_out(ci, p):
            pltpu.async_copy(
                nar[p],
                out_hbm.at[pl.ds((wid * bpw + ci) * SEQ, SEQ)],
                osem[p],
            )

        def drain_out(p):
            pltpu.make_async_copy(
                nar[p], out_hbm.at[pl.ds(0, SEQ)], osem[p]
            ).wait()

        def scale(p):
            src, dst = wide[p], nar[p]

            def srows(r, carry):
                base = r * RUNROLL
                for k in range(RUNROLL):
                    for l in range(D_MODEL // 16):
                        dst[base + k, pl.ds(l * 16, 16)] = (
                            src[base + k, pl.ds(l * 16, 16)] * SCALE
                        )
                return carry

            lax.fori_loop(0, SEQ // RUNROLL, srows, 0, unroll=False)

        load_group(0)
        fire(0, 0)

        def pair_body(i, carry):
            for p in (0, 1):
                ci = 2 * i + p
                nci = ci + 1

                @pl.when(jnp.logical_and(nci < bpw, nci % GRPB != 0))
                def _():
                    fire(nci, 1 - p)

                drain_gathers(p)

                @pl.when(jnp.logical_and(nci < bpw, nci % GRPB == 0))
                def _():
                    load_group(nci // GRPB)
                    fire(nci, 1 - p)

                @pl.when(ci >= 2)
                def _():
                    drain_out(p)

                scale(p)
                put_out(ci, p)
            return carry

        lax.fori_loop(0, bpw // 2, pair_body, 0, unroll=False)
        drain_out(0)
        drain_out(1)

    return gather_kernel


def kernel(x, table):
    BATCH = x.shape[0]
    V = table.shape[0]
    xp = jnp.pad(x, ((0, 0), (0, SEQP - SEQ)))
    table_pad = jnp.pad(table, ((0, 0), (0, PADW - D_MODEL)))
    out2d = _build_gather(BATCH, V)(xp, table_pad)
    return out2d.reshape(BATCH, SEQ, D_MODEL)
